# baseline (speedup 1.0000x reference)
import functools

import jax
import jax.numpy as jnp
from jax.experimental import pallas as pl
from jax.experimental.pallas import tpu as pltpu
from jax.experimental.pallas import tpu_sc as plsc

_MIN_CONF = 0.7
_MAX_INST = 100
_NMS_THR = 0.3

_TILE = 800
_LANES = 128


def _phase_a_kernel(probs_ref, s_ref, c_ref, w_ref, l_ref):
    p = probs_ref[...]
    m = jnp.max(p, axis=1, keepdims=True)
    ncls = p.shape[1]
    col = jax.lax.broadcasted_iota(jnp.int32, p.shape, 1)
    cid = jnp.min(jnp.where(p == m, col, ncls), axis=1, keepdims=True)
    s_ref[...] = m
    c_ref[...] = cid.astype(jnp.float32)
    tile = p.shape[0]
    row = (pl.program_id(0) * tile
           + jax.lax.broadcasted_iota(jnp.int32, (tile, 1), 0))
    flat = row * (4 * ncls) + cid * 4
    w_ref[...] = flat >> 7
    l_ref[...] = flat & 127


def _sc_gather(table2d, widx, lidx):
    b = widx.shape[0]
    info = plsc.get_sparse_core_info()
    nw = info.num_cores * info.num_subcores
    bpw = b // nw
    nchunk = bpw // 16
    mesh = plsc.VectorSubcoreMesh(core_axis_name="c", subcore_axis_name="s")

    @functools.partial(
        pl.kernel,
        out_type=jax.ShapeDtypeStruct((4, b), jnp.float32),
        mesh=mesh,
        scratch_types=[
            pltpu.VMEM((bpw,), jnp.int32),
            pltpu.VMEM((bpw,), jnp.int32),
            pltpu.VMEM((bpw, 128), jnp.float32),
            pltpu.VMEM((4, bpw), jnp.float32),
            pltpu.SemaphoreType.DMA,
        ],
        compiler_params=pltpu.CompilerParams(needs_layout_passes=False),
    )
    def k(table_hbm, widx_hbm, lidx_hbm, out_hbm, widx_v, lidx_v, rows_v,
          out_v, sem):
        wid = jax.lax.axis_index("s") * info.num_cores + jax.lax.axis_index("c")
        base = wid * bpw
        pltpu.sync_copy(widx_hbm.at[pl.ds(base, bpw)], widx_v)
        pltpu.sync_copy(lidx_hbm.at[pl.ds(base, bpw)], lidx_v)
        pltpu.async_copy(table_hbm.at[widx_v], rows_v, sem).wait()

        def body(j, carry):
            rid = jax.lax.iota(jnp.int32, 16) + j * 16
            lane = lidx_v[pl.ds(j * 16, 16)]
            for kk in range(4):
                out_v[kk, pl.ds(j * 16, 16)] = plsc.load_gather(
                    rows_v, [rid, lane + kk])
            return carry

        jax.lax.fori_loop(0, nchunk, body, 0)
        pltpu.sync_copy(out_v, out_hbm.at[:, pl.ds(base, bpw)])

    return k(table2d, widx, lidx)


def _phase_b_kernel(n_total, rois_ref, dsel_ref, cid_ref, score_ref, win_ref,
                    out_ref, y1s, x1s, y2s, x2s):
    wy1 = win_ref[0]
    wx1 = win_ref[1]
    wy2 = win_ref[2]
    wx2 = win_ref[3]
    y1 = rois_ref[0]
    x1 = rois_ref[1]
    y2 = rois_ref[2]
    x2 = rois_ref[3]
    d0 = dsel_ref[0] * 0.1
    d1 = dsel_ref[1] * 0.1
    d2 = dsel_ref[2] * 0.2
    d3 = dsel_ref[3] * 0.2
    h = y2 - y1
    w = x2 - x1
    cy = y1 + 0.5 * h + d0 * h
    cx = x1 + 0.5 * w + d1 * w
    h = h * jnp.exp(d2)
    w = w * jnp.exp(d3)
    ry1 = cy - 0.5 * h
    rx1 = cx - 0.5 * w
    ry2 = ry1 + h
    rx2 = rx1 + w
    ry1 = jnp.clip(ry1, wy1, wy2)
    rx1 = jnp.clip(rx1, wx1, wx2)
    ry2 = jnp.clip(ry2, wy1, wy2)
    rx2 = jnp.clip(rx2, wx1, wx2)
    y1s[...] = ry1
    x1s[...] = rx1
    y2s[...] = ry2
    x2s[...] = rx2

    cidf = cid_ref[...]
    scr = score_ref[...]
    rowi = jax.lax.broadcasted_iota(jnp.int32, scr.shape, 0)
    coli = jax.lax.broadcasted_iota(jnp.int32, scr.shape, 1)
    flat = rowi * _LANES + coli
    in_range = flat < n_total
    keep = in_range & (cidf > 0.5) & (scr >= _MIN_CONF)
    scores0 = jnp.where(keep, scr, -1.0)

    off = cidf * 4.0
    ny1 = ry1 + off
    nx1 = rx1 + off
    ny2 = ry2 + off
    nx2 = rx2 + off
    areas = (ny2 - ny1) * (nx2 - nx1)
    lane = jax.lax.broadcasted_iota(jnp.int32, (1, _LANES), 1)
    zero_lane = jnp.zeros((1, _LANES), jnp.float32)

    def body(i, carry):
        scores, by1, bx1, by2, bx2, bcl, bsc = carry
        m = jnp.max(scores)
        idx = jnp.min(jnp.where(scores == m, flat, jnp.int32(1 << 30)))
        r = idx >> 7
        c = idx & (_LANES - 1)
        laneeq = lane == c

        def ext(ref):
            return jnp.sum(jnp.where(laneeq, ref[pl.ds(r, 1), :], 0.0))

        ey1 = ext(y1s)
        ex1 = ext(x1s)
        ey2 = ext(y2s)
        ex2 = ext(x2s)
        ecl = ext(cid_ref)
        o = ecl * 4.0
        a1 = ey1 + o
        a2 = ey2 + o
        b1 = ex1 + o
        b2 = ex2 + o
        yy1 = jnp.maximum(a1, ny1)
        xx1 = jnp.maximum(b1, nx1)
        yy2 = jnp.minimum(a2, ny2)
        xx2 = jnp.minimum(b2, nx2)
        inter = jnp.maximum(yy2 - yy1, 0.0) * jnp.maximum(xx2 - xx1, 0.0)
        union = (a2 - a1) * (b2 - b1) + areas - inter
        iou = inter / (union + 1e-8)
        supp = (iou > _NMS_THR) | (flat == idx)
        scores = jnp.where(supp, -1.0, scores)
        li = lane == i
        by1 = jnp.where(li, ey1, by1)
        bx1 = jnp.where(li, ex1, bx1)
        by2 = jnp.where(li, ey2, by2)
        bx2 = jnp.where(li, ex2, bx2)
        bcl = jnp.where(li, ecl, bcl)
        bsc = jnp.where(li, m, bsc)
        return scores, by1, bx1, by2, bx2, bcl, bsc

    init = (scores0, zero_lane, zero_lane, zero_lane, zero_lane, zero_lane,
            zero_lane)
    _, by1, bx1, by2, bx2, bcl, bsc = jax.lax.fori_loop(
        0, _MAX_INST, body, init)
    valid = bsc > 0.0
    vf = valid.astype(jnp.float32)
    out_ref[...] = jnp.concatenate(
        [by1 * vf, bx1 * vf, by2 * vf, bx2 * vf, bcl * vf,
         jnp.where(valid, bsc, 0.0), zero_lane, zero_lane], axis=0)


def kernel(rois, probs, deltas, window):
    n, ncls = probs.shape
    grid = n // _TILE
    s2, c2, w2, l2 = pl.pallas_call(
        _phase_a_kernel,
        grid=(grid,),
        in_specs=[
            pl.BlockSpec((_TILE, ncls), lambda i: (i, 0)),
        ],
        out_specs=[
            pl.BlockSpec((_TILE, 1), lambda i: (i, 0)),
            pl.BlockSpec((_TILE, 1), lambda i: (i, 0)),
            pl.BlockSpec((_TILE, 1), lambda i: (i, 0)),
            pl.BlockSpec((_TILE, 1), lambda i: (i, 0)),
        ],
        out_shape=[
            jax.ShapeDtypeStruct((n, 1), jnp.float32),
            jax.ShapeDtypeStruct((n, 1), jnp.float32),
            jax.ShapeDtypeStruct((n, 1), jnp.int32),
            jax.ShapeDtypeStruct((n, 1), jnp.int32),
        ],
    )(probs)

    sc_b = -(-n // 512) * 512
    wpad = jnp.pad(w2[:, 0], (0, sc_b - n))
    lpad = jnp.pad(l2[:, 0], (0, sc_b - n))
    d4pl = _sc_gather(deltas.reshape(-1, _LANES), wpad, lpad)

    n_pad = -(-n // _LANES) * _LANES
    rows = n_pad // _LANES
    pad = n_pad - n

    def plane(x):
        return jnp.pad(x[:, 0], (0, pad)).reshape(rows, _LANES)

    roisp = jnp.pad(rois, ((0, pad), (0, 0))).T.reshape(4, rows, _LANES)
    dselp = d4pl[:, :n_pad].reshape(4, rows, _LANES)

    det = pl.pallas_call(
        functools.partial(_phase_b_kernel, n),
        in_specs=[
            pl.BlockSpec(memory_space=pltpu.VMEM),
            pl.BlockSpec(memory_space=pltpu.VMEM),
            pl.BlockSpec(memory_space=pltpu.VMEM),
            pl.BlockSpec(memory_space=pltpu.VMEM),
            pl.BlockSpec(memory_space=pltpu.SMEM),
        ],
        out_shape=jax.ShapeDtypeStruct((8, _LANES), jnp.float32),
        scratch_shapes=[pltpu.VMEM((rows, _LANES), jnp.float32)] * 4,
    )(roisp, dselp, plane(c2), plane(s2), window)

    boxes = det[0:4, :_MAX_INST].T
    cls = det[4:5, :_MAX_INST].T
    sc = det[5:6, :_MAX_INST].T
    return jnp.concatenate([boxes, cls, sc], axis=1)


# speedup vs baseline: 9.2637x; 9.2637x over previous
import functools

import jax
import jax.numpy as jnp
from jax.experimental import pallas as pl
from jax.experimental.pallas import tpu as pltpu
from jax.experimental.pallas import tpu_sc as plsc

_MIN_CONF = 0.7
_MAX_INST = 100
_NMS_THR = 0.3

_TILE = 800
_LANES = 128


def _phase_a_kernel(probs_ref, s_ref, c_ref, w_ref, l_ref):
    p = probs_ref[...]
    m = jnp.max(p, axis=1, keepdims=True)
    ncls = p.shape[1]
    col = jax.lax.broadcasted_iota(jnp.int32, p.shape, 1)
    cid = jnp.min(jnp.where(p == m, col, ncls), axis=1, keepdims=True)
    s_ref[...] = m
    c_ref[...] = cid.astype(jnp.float32)
    tile = p.shape[0]
    row = (pl.program_id(0) * tile
           + jax.lax.broadcasted_iota(jnp.int32, (tile, 1), 0))
    n_total = pl.num_programs(0) * tile
    bank = cid >> 5
    w_ref[...] = bank * n_total + row
    l_ref[...] = (cid * 4) & 127


def _repack_kernel(din_ref, dout_ref):
    dout_ref[...] = din_ref[...]


def _sc_gather(table2d, widx, lidx):
    b = widx.shape[0]
    info = plsc.get_sparse_core_info()
    nw = info.num_cores * info.num_subcores
    bpw = b // nw
    nchunk = bpw // 16
    mesh = plsc.VectorSubcoreMesh(core_axis_name="c", subcore_axis_name="s")

    @functools.partial(
        pl.kernel,
        out_type=jax.ShapeDtypeStruct((4, b), jnp.float32),
        mesh=mesh,
        scratch_types=[
            pltpu.VMEM((bpw,), jnp.int32),
            pltpu.VMEM((bpw,), jnp.int32),
            pltpu.VMEM((bpw, 128), jnp.float32),
            pltpu.VMEM((4, bpw), jnp.float32),
            pltpu.SemaphoreType.DMA,
        ],
        compiler_params=pltpu.CompilerParams(needs_layout_passes=False),
    )
    def k(table_hbm, widx_hbm, lidx_hbm, out_hbm, widx_v, lidx_v, rows_v,
          out_v, sem):
        wid = jax.lax.axis_index("s") * info.num_cores + jax.lax.axis_index("c")
        base = wid * bpw
        pltpu.sync_copy(widx_hbm.at[pl.ds(base, bpw)], widx_v)
        pltpu.sync_copy(lidx_hbm.at[pl.ds(base, bpw)], lidx_v)
        pltpu.async_copy(table_hbm.at[widx_v], rows_v, sem).wait()

        def body(j, carry):
            rid = jax.lax.iota(jnp.int32, 16) + j * 16
            lane = lidx_v[pl.ds(j * 16, 16)]
            for kk in range(4):
                out_v[kk, pl.ds(j * 16, 16)] = plsc.load_gather(
                    rows_v, [rid, lane + kk])
            return carry

        jax.lax.fori_loop(0, nchunk, body, 0)
        pltpu.sync_copy(out_v, out_hbm.at[:, pl.ds(base, bpw)])

    return k(table2d, widx, lidx)


def _phase_b_kernel(n_total, rois_ref, dsel_ref, cid_ref, score_ref, win_ref,
                    out_ref, y1s, x1s, y2s, x2s):
    wy1 = win_ref[0]
    wx1 = win_ref[1]
    wy2 = win_ref[2]
    wx2 = win_ref[3]
    y1 = rois_ref[0]
    x1 = rois_ref[1]
    y2 = rois_ref[2]
    x2 = rois_ref[3]
    d0 = dsel_ref[0] * 0.1
    d1 = dsel_ref[1] * 0.1
    d2 = dsel_ref[2] * 0.2
    d3 = dsel_ref[3] * 0.2
    h = y2 - y1
    w = x2 - x1
    cy = y1 + 0.5 * h + d0 * h
    cx = x1 + 0.5 * w + d1 * w
    h = h * jnp.exp(d2)
    w = w * jnp.exp(d3)
    ry1 = cy - 0.5 * h
    rx1 = cx - 0.5 * w
    ry2 = ry1 + h
    rx2 = rx1 + w
    ry1 = jnp.clip(ry1, wy1, wy2)
    rx1 = jnp.clip(rx1, wx1, wx2)
    ry2 = jnp.clip(ry2, wy1, wy2)
    rx2 = jnp.clip(rx2, wx1, wx2)
    y1s[...] = ry1
    x1s[...] = rx1
    y2s[...] = ry2
    x2s[...] = rx2

    cidf = cid_ref[...]
    scr = score_ref[...]
    rowi = jax.lax.broadcasted_iota(jnp.int32, scr.shape, 0)
    coli = jax.lax.broadcasted_iota(jnp.int32, scr.shape, 1)
    flat = rowi * _LANES + coli
    in_range = flat < n_total
    keep = in_range & (cidf > 0.5) & (scr >= _MIN_CONF)
    scores0 = jnp.where(keep, scr, -1.0)

    off = cidf * 4.0
    ny1 = ry1 + off
    nx1 = rx1 + off
    ny2 = ry2 + off
    nx2 = rx2 + off
    areas = (ny2 - ny1) * (nx2 - nx1)
    lane = jax.lax.broadcasted_iota(jnp.int32, (1, _LANES), 1)
    zero_lane = jnp.zeros((1, _LANES), jnp.float32)

    def body(i, carry):
        scores, by1, bx1, by2, bx2, bcl, bsc = carry
        m = jnp.max(scores)
        idx = jnp.min(jnp.where(scores == m, flat, jnp.int32(1 << 30)))
        r = idx >> 7
        c = idx & (_LANES - 1)
        laneeq = lane == c

        def ext(ref):
            return jnp.sum(jnp.where(laneeq, ref[pl.ds(r, 1), :], 0.0))

        ey1 = ext(y1s)
        ex1 = ext(x1s)
        ey2 = ext(y2s)
        ex2 = ext(x2s)
        ecl = ext(cid_ref)
        o = ecl * 4.0
        a1 = ey1 + o
        a2 = ey2 + o
        b1 = ex1 + o
        b2 = ex2 + o
        yy1 = jnp.maximum(a1, ny1)
        xx1 = jnp.maximum(b1, nx1)
        yy2 = jnp.minimum(a2, ny2)
        xx2 = jnp.minimum(b2, nx2)
        inter = jnp.maximum(yy2 - yy1, 0.0) * jnp.maximum(xx2 - xx1, 0.0)
        union = (a2 - a1) * (b2 - b1) + areas - inter
        iou = inter / (union + 1e-8)
        supp = (iou > _NMS_THR) | (flat == idx)
        scores = jnp.where(supp, -1.0, scores)
        li = lane == i
        by1 = jnp.where(li, ey1, by1)
        bx1 = jnp.where(li, ex1, bx1)
        by2 = jnp.where(li, ey2, by2)
        bx2 = jnp.where(li, ex2, bx2)
        bcl = jnp.where(li, ecl, bcl)
        bsc = jnp.where(li, m, bsc)
        return scores, by1, bx1, by2, bx2, bcl, bsc

    init = (scores0, zero_lane, zero_lane, zero_lane, zero_lane, zero_lane,
            zero_lane)
    _, by1, bx1, by2, bx2, bcl, bsc = jax.lax.fori_loop(
        0, _MAX_INST, body, init)
    valid = bsc > 0.0
    vf = valid.astype(jnp.float32)
    out_ref[...] = jnp.concatenate(
        [by1 * vf, bx1 * vf, by2 * vf, bx2 * vf, bcl * vf,
         jnp.where(valid, bsc, 0.0), zero_lane, zero_lane], axis=0)


def kernel(rois, probs, deltas, window):
    n, ncls = probs.shape
    grid = n // _TILE
    s2, c2, w2, l2 = pl.pallas_call(
        _phase_a_kernel,
        grid=(grid,),
        in_specs=[
            pl.BlockSpec((_TILE, ncls), lambda i: (i, 0)),
        ],
        out_specs=[
            pl.BlockSpec((_TILE, 1), lambda i: (i, 0)),
            pl.BlockSpec((_TILE, 1), lambda i: (i, 0)),
            pl.BlockSpec((_TILE, 1), lambda i: (i, 0)),
            pl.BlockSpec((_TILE, 1), lambda i: (i, 0)),
        ],
        out_shape=[
            jax.ShapeDtypeStruct((n, 1), jnp.float32),
            jax.ShapeDtypeStruct((n, 1), jnp.float32),
            jax.ShapeDtypeStruct((n, 1), jnp.int32),
            jax.ShapeDtypeStruct((n, 1), jnp.int32),
        ],
    )(probs)

    rp_tile = 2000
    rp_grid = n // rp_tile
    nbanks = -(-4 * ncls // _LANES)
    table = pl.pallas_call(
        _repack_kernel,
        grid=(nbanks, rp_grid),
        in_specs=[pl.BlockSpec((rp_tile, _LANES), lambda b, i: (i, b))],
        out_specs=pl.BlockSpec(
            (rp_tile, _LANES), lambda b, i: (b * rp_grid + i, 0)),
        out_shape=jax.ShapeDtypeStruct((nbanks * n, _LANES), jnp.float32),
    )(deltas.reshape(n, 4 * ncls))

    sc_b = -(-n // 512) * 512
    wpad = jnp.pad(w2[:, 0], (0, sc_b - n))
    lpad = jnp.pad(l2[:, 0], (0, sc_b - n))
    d4pl = _sc_gather(table, wpad, lpad)

    n_pad = -(-n // _LANES) * _LANES
    rows = n_pad // _LANES
    pad = n_pad - n

    def plane(x):
        return jnp.pad(x[:, 0], (0, pad)).reshape(rows, _LANES)

    roisp = jnp.pad(rois, ((0, pad), (0, 0))).T.reshape(4, rows, _LANES)
    dselp = d4pl[:, :n_pad].reshape(4, rows, _LANES)

    det = pl.pallas_call(
        functools.partial(_phase_b_kernel, n),
        in_specs=[
            pl.BlockSpec(memory_space=pltpu.VMEM),
            pl.BlockSpec(memory_space=pltpu.VMEM),
            pl.BlockSpec(memory_space=pltpu.VMEM),
            pl.BlockSpec(memory_space=pltpu.VMEM),
            pl.BlockSpec(memory_space=pltpu.SMEM),
        ],
        out_shape=jax.ShapeDtypeStruct((8, _LANES), jnp.float32),
        scratch_shapes=[pltpu.VMEM((rows, _LANES), jnp.float32)] * 4,
    )(roisp, dselp, plane(c2), plane(s2), window)

    boxes = det[0:4, :_MAX_INST].T
    cls = det[4:5, :_MAX_INST].T
    sc = det[5:6, :_MAX_INST].T
    return jnp.concatenate([boxes, cls, sc], axis=1)


# speedup vs baseline: 11.2926x; 1.2190x over previous
import functools

import jax
import jax.numpy as jnp
from jax.experimental import pallas as pl
from jax.experimental.pallas import tpu as pltpu
from jax.experimental.pallas import tpu_sc as plsc

_MIN_CONF = 0.7
_MAX_INST = 100
_NMS_THR = 0.3

_TILE = 800
_LANES = 128


def _phase_a_kernel(probs_ref, deltas_ref, s_ref, c_ref, l_ref, win_out_ref):
    p = probs_ref[...]
    m = jnp.max(p, axis=1, keepdims=True)
    ncls = p.shape[1]
    col = jax.lax.broadcasted_iota(jnp.int32, p.shape, 1)
    cid = jnp.min(jnp.where(p == m, col, ncls), axis=1, keepdims=True)
    s_ref[...] = m
    c_ref[...] = cid.astype(jnp.float32)
    l_ref[...] = (cid * 4) & 127
    bank = cid >> 5
    d = deltas_ref[...]
    w = jnp.where(bank == 0, d[:, 0:128],
                  jnp.where(bank == 1, d[:, 128:256], d[:, 256:384]))
    win_out_ref[...] = w


def _sc_extract(windows, lidx):
    b = windows.shape[0]
    info = plsc.get_sparse_core_info()
    nw = info.num_cores * info.num_subcores
    bpw = b // nw
    nchunk = bpw // 16
    mesh = plsc.VectorSubcoreMesh(core_axis_name="c", subcore_axis_name="s")

    @functools.partial(
        pl.kernel,
        out_type=jax.ShapeDtypeStruct((4, b), jnp.float32),
        mesh=mesh,
        scratch_types=[
            pltpu.VMEM((bpw,), jnp.int32),
            pltpu.VMEM((bpw, 128), jnp.float32),
            pltpu.VMEM((4, bpw), jnp.float32),
            pltpu.SemaphoreType.DMA,
        ],
        compiler_params=pltpu.CompilerParams(needs_layout_passes=False),
    )
    def k(win_hbm, lidx_hbm, out_hbm, lidx_v, rows_v, out_v, sem):
        wid = jax.lax.axis_index("s") * info.num_cores + jax.lax.axis_index("c")
        base = wid * bpw
        pltpu.sync_copy(lidx_hbm.at[pl.ds(base, bpw)], lidx_v)
        pltpu.async_copy(win_hbm.at[pl.ds(base, bpw)], rows_v, sem).wait()

        def body(j, carry):
            rid = jax.lax.iota(jnp.int32, 16) + j * 16
            lane = lidx_v[pl.ds(j * 16, 16)]
            for kk in range(4):
                out_v[kk, pl.ds(j * 16, 16)] = plsc.load_gather(
                    rows_v, [rid, lane + kk])
            return carry

        jax.lax.fori_loop(0, nchunk, body, 0)
        pltpu.sync_copy(out_v, out_hbm.at[:, pl.ds(base, bpw)])

    return k(windows, lidx)


def _phase_b_kernel(n_total, rois_ref, dsel_ref, cid_ref, score_ref, win_ref,
                    out_ref, y1s, x1s, y2s, x2s):
    wy1 = win_ref[0]
    wx1 = win_ref[1]
    wy2 = win_ref[2]
    wx2 = win_ref[3]
    y1 = rois_ref[0]
    x1 = rois_ref[1]
    y2 = rois_ref[2]
    x2 = rois_ref[3]
    d0 = dsel_ref[0] * 0.1
    d1 = dsel_ref[1] * 0.1
    d2 = dsel_ref[2] * 0.2
    d3 = dsel_ref[3] * 0.2
    h = y2 - y1
    w = x2 - x1
    cy = y1 + 0.5 * h + d0 * h
    cx = x1 + 0.5 * w + d1 * w
    h = h * jnp.exp(d2)
    w = w * jnp.exp(d3)
    ry1 = cy - 0.5 * h
    rx1 = cx - 0.5 * w
    ry2 = ry1 + h
    rx2 = rx1 + w
    ry1 = jnp.clip(ry1, wy1, wy2)
    rx1 = jnp.clip(rx1, wx1, wx2)
    ry2 = jnp.clip(ry2, wy1, wy2)
    rx2 = jnp.clip(rx2, wx1, wx2)
    y1s[...] = ry1
    x1s[...] = rx1
    y2s[...] = ry2
    x2s[...] = rx2

    cidf = cid_ref[...]
    scr = score_ref[...]
    rowi = jax.lax.broadcasted_iota(jnp.int32, scr.shape, 0)
    coli = jax.lax.broadcasted_iota(jnp.int32, scr.shape, 1)
    flat = rowi * _LANES + coli
    in_range = flat < n_total
    keep = in_range & (cidf > 0.5) & (scr >= _MIN_CONF)
    scores0 = jnp.where(keep, scr, -1.0)

    off = cidf * 4.0
    ny1 = ry1 + off
    nx1 = rx1 + off
    ny2 = ry2 + off
    nx2 = rx2 + off
    areas = (ny2 - ny1) * (nx2 - nx1)
    lane = jax.lax.broadcasted_iota(jnp.int32, (1, _LANES), 1)
    zero_lane = jnp.zeros((1, _LANES), jnp.float32)

    def body(i, carry):
        scores, by1, bx1, by2, bx2, bcl, bsc = carry
        m = jnp.max(scores)
        idx = jnp.min(jnp.where(scores == m, flat, jnp.int32(1 << 30)))
        r = idx >> 7
        c = idx & (_LANES - 1)
        laneeq = lane == c

        def ext(ref):
            return jnp.sum(jnp.where(laneeq, ref[pl.ds(r, 1), :], 0.0))

        ey1 = ext(y1s)
        ex1 = ext(x1s)
        ey2 = ext(y2s)
        ex2 = ext(x2s)
        ecl = ext(cid_ref)
        o = ecl * 4.0
        a1 = ey1 + o
        a2 = ey2 + o
        b1 = ex1 + o
        b2 = ex2 + o
        yy1 = jnp.maximum(a1, ny1)
        xx1 = jnp.maximum(b1, nx1)
        yy2 = jnp.minimum(a2, ny2)
        xx2 = jnp.minimum(b2, nx2)
        inter = jnp.maximum(yy2 - yy1, 0.0) * jnp.maximum(xx2 - xx1, 0.0)
        union = (a2 - a1) * (b2 - b1) + areas - inter
        iou = inter / (union + 1e-8)
        supp = (iou > _NMS_THR) | (flat == idx)
        scores = jnp.where(supp, -1.0, scores)
        li = lane == i
        by1 = jnp.where(li, ey1, by1)
        bx1 = jnp.where(li, ex1, bx1)
        by2 = jnp.where(li, ey2, by2)
        bx2 = jnp.where(li, ex2, bx2)
        bcl = jnp.where(li, ecl, bcl)
        bsc = jnp.where(li, m, bsc)
        return scores, by1, bx1, by2, bx2, bcl, bsc

    init = (scores0, zero_lane, zero_lane, zero_lane, zero_lane, zero_lane,
            zero_lane)
    _, by1, bx1, by2, bx2, bcl, bsc = jax.lax.fori_loop(
        0, _MAX_INST, body, init)
    valid = bsc > 0.0
    vf = valid.astype(jnp.float32)
    out_ref[...] = jnp.concatenate(
        [by1 * vf, bx1 * vf, by2 * vf, bx2 * vf, bcl * vf,
         jnp.where(valid, bsc, 0.0), zero_lane, zero_lane], axis=0)


def kernel(rois, probs, deltas, window):
    n, ncls = probs.shape
    grid = n // _TILE
    sc_b = -(-n // 512) * 512
    nbanks = -(-4 * ncls // _LANES)
    s2, c2, l2, windows = pl.pallas_call(
        _phase_a_kernel,
        grid=(grid,),
        in_specs=[
            pl.BlockSpec((_TILE, ncls), lambda i: (i, 0)),
            pl.BlockSpec((_TILE, nbanks * _LANES), lambda i: (i, 0)),
        ],
        out_specs=[
            pl.BlockSpec((_TILE, 1), lambda i: (i, 0)),
            pl.BlockSpec((_TILE, 1), lambda i: (i, 0)),
            pl.BlockSpec((_TILE, 1), lambda i: (i, 0)),
            pl.BlockSpec((_TILE, _LANES), lambda i: (i, 0)),
        ],
        out_shape=[
            jax.ShapeDtypeStruct((n, 1), jnp.float32),
            jax.ShapeDtypeStruct((n, 1), jnp.float32),
            jax.ShapeDtypeStruct((n, 1), jnp.int32),
            jax.ShapeDtypeStruct((sc_b, _LANES), jnp.float32),
        ],
    )(probs, deltas.reshape(n, 4 * ncls))

    lpad = jnp.pad(l2[:, 0], (0, sc_b - n))
    d4pl = _sc_extract(windows, lpad)

    n_pad = -(-n // _LANES) * _LANES
    rows = n_pad // _LANES
    pad = n_pad - n

    def plane(x):
        return jnp.pad(x[:, 0], (0, pad)).reshape(rows, _LANES)

    roisp = jnp.pad(rois, ((0, pad), (0, 0))).T.reshape(4, rows, _LANES)
    dselp = d4pl[:, :n_pad].reshape(4, rows, _LANES)

    det = pl.pallas_call(
        functools.partial(_phase_b_kernel, n),
        in_specs=[
            pl.BlockSpec(memory_space=pltpu.VMEM),
            pl.BlockSpec(memory_space=pltpu.VMEM),
            pl.BlockSpec(memory_space=pltpu.VMEM),
            pl.BlockSpec(memory_space=pltpu.VMEM),
            pl.BlockSpec(memory_space=pltpu.SMEM),
        ],
        out_shape=jax.ShapeDtypeStruct((8, _LANES), jnp.float32),
        scratch_shapes=[pltpu.VMEM((rows, _LANES), jnp.float32)] * 4,
    )(roisp, dselp, plane(c2), plane(s2), window)

    boxes = det[0:4, :_MAX_INST].T
    cls = det[4:5, :_MAX_INST].T
    sc = det[5:6, :_MAX_INST].T
    return jnp.concatenate([boxes, cls, sc], axis=1)


# speedup vs baseline: 11.7021x; 1.0363x over previous
import functools

import jax
import jax.numpy as jnp
from jax.experimental import pallas as pl
from jax.experimental.pallas import tpu as pltpu
from jax.experimental.pallas import tpu_sc as plsc

_MIN_CONF = 0.7
_MAX_INST = 100
_NMS_THR = 0.3

_TILE = 2000
_LANES = 128


def _phase_a_kernel(probs_ref, deltas_ref, s_ref, c_ref, l_ref, win_out_ref):
    p = probs_ref[...]
    m = jnp.max(p, axis=1, keepdims=True)
    ncls = p.shape[1]
    col = jax.lax.broadcasted_iota(jnp.int32, p.shape, 1)
    cid = jnp.min(jnp.where(p == m, col, ncls), axis=1, keepdims=True)
    s_ref[...] = m
    c_ref[...] = cid.astype(jnp.float32)
    l_ref[...] = (cid * 4) & 127
    bank = cid >> 5
    d = deltas_ref[...]
    w = jnp.where(bank == 0, d[:, 0:128],
                  jnp.where(bank == 1, d[:, 128:256], d[:, 256:384]))
    win_out_ref[...] = w


def _sc_extract(windows, lidx):
    b = windows.shape[0]
    info = plsc.get_sparse_core_info()
    nw = info.num_cores * info.num_subcores
    bpw = b // nw
    nchunk = bpw // 16
    mesh = plsc.VectorSubcoreMesh(core_axis_name="c", subcore_axis_name="s")

    @functools.partial(
        pl.kernel,
        out_type=jax.ShapeDtypeStruct((4, b), jnp.float32),
        mesh=mesh,
        scratch_types=[
            pltpu.VMEM((bpw,), jnp.int32),
            pltpu.VMEM((bpw, 128), jnp.float32),
            pltpu.VMEM((4, bpw), jnp.float32),
            pltpu.SemaphoreType.DMA,
        ],
        compiler_params=pltpu.CompilerParams(needs_layout_passes=False),
    )
    def k(win_hbm, lidx_hbm, out_hbm, lidx_v, rows_v, out_v, sem):
        wid = jax.lax.axis_index("s") * info.num_cores + jax.lax.axis_index("c")
        base = wid * bpw
        pltpu.sync_copy(lidx_hbm.at[pl.ds(base, bpw)], lidx_v)
        pltpu.async_copy(win_hbm.at[pl.ds(base, bpw)], rows_v, sem).wait()

        def body(j, carry):
            rid = jax.lax.iota(jnp.int32, 16) + j * 16
            lane = lidx_v[pl.ds(j * 16, 16)]
            for kk in range(4):
                out_v[kk, pl.ds(j * 16, 16)] = plsc.load_gather(
                    rows_v, [rid, lane + kk])
            return carry

        jax.lax.fori_loop(0, nchunk, body, 0)
        pltpu.sync_copy(out_v, out_hbm.at[:, pl.ds(base, bpw)])

    return k(windows, lidx)


def _phase_b_kernel(n_total, rois_ref, dsel_ref, cid_ref, score_ref, win_ref,
                    out_ref, y1s, x1s, y2s, x2s):
    wy1 = win_ref[0]
    wx1 = win_ref[1]
    wy2 = win_ref[2]
    wx2 = win_ref[3]
    y1 = rois_ref[0]
    x1 = rois_ref[1]
    y2 = rois_ref[2]
    x2 = rois_ref[3]
    d0 = dsel_ref[0] * 0.1
    d1 = dsel_ref[1] * 0.1
    d2 = dsel_ref[2] * 0.2
    d3 = dsel_ref[3] * 0.2
    h = y2 - y1
    w = x2 - x1
    cy = y1 + 0.5 * h + d0 * h
    cx = x1 + 0.5 * w + d1 * w
    h = h * jnp.exp(d2)
    w = w * jnp.exp(d3)
    ry1 = cy - 0.5 * h
    rx1 = cx - 0.5 * w
    ry2 = ry1 + h
    rx2 = rx1 + w
    ry1 = jnp.clip(ry1, wy1, wy2)
    rx1 = jnp.clip(rx1, wx1, wx2)
    ry2 = jnp.clip(ry2, wy1, wy2)
    rx2 = jnp.clip(rx2, wx1, wx2)
    y1s[...] = ry1
    x1s[...] = rx1
    y2s[...] = ry2
    x2s[...] = rx2

    cidf = cid_ref[...]
    scr = score_ref[...]
    rowi = jax.lax.broadcasted_iota(jnp.int32, scr.shape, 0)
    coli = jax.lax.broadcasted_iota(jnp.int32, scr.shape, 1)
    flat = rowi * _LANES + coli
    in_range = flat < n_total
    keep = in_range & (cidf > 0.5) & (scr >= _MIN_CONF)
    scores0 = jnp.where(keep, scr, -1.0)

    off = cidf * 4.0
    ny1 = ry1 + off
    nx1 = rx1 + off
    ny2 = ry2 + off
    nx2 = rx2 + off
    areas = (ny2 - ny1) * (nx2 - nx1)
    lane = jax.lax.broadcasted_iota(jnp.int32, (1, _LANES), 1)
    zero_lane = jnp.zeros((1, _LANES), jnp.float32)

    def body(i, carry):
        scores, by1, bx1, by2, bx2, bcl, bsc = carry
        m = jnp.max(scores)
        idx = jnp.min(jnp.where(scores == m, flat, jnp.int32(1 << 30)))
        r = idx >> 7
        c = idx & (_LANES - 1)
        laneeq = lane == c

        def ext(ref):
            return jnp.sum(jnp.where(laneeq, ref[pl.ds(r, 1), :], 0.0))

        ey1 = ext(y1s)
        ex1 = ext(x1s)
        ey2 = ext(y2s)
        ex2 = ext(x2s)
        ecl = ext(cid_ref)
        o = ecl * 4.0
        a1 = ey1 + o
        a2 = ey2 + o
        b1 = ex1 + o
        b2 = ex2 + o
        yy1 = jnp.maximum(a1, ny1)
        xx1 = jnp.maximum(b1, nx1)
        yy2 = jnp.minimum(a2, ny2)
        xx2 = jnp.minimum(b2, nx2)
        inter = jnp.maximum(yy2 - yy1, 0.0) * jnp.maximum(xx2 - xx1, 0.0)
        union = (a2 - a1) * (b2 - b1) + areas - inter
        iou = inter / (union + 1e-8)
        supp = (iou > _NMS_THR) | (flat == idx)
        scores = jnp.where(supp, -1.0, scores)
        li = lane == i
        by1 = jnp.where(li, ey1, by1)
        bx1 = jnp.where(li, ex1, bx1)
        by2 = jnp.where(li, ey2, by2)
        bx2 = jnp.where(li, ex2, bx2)
        bcl = jnp.where(li, ecl, bcl)
        bsc = jnp.where(li, m, bsc)
        return scores, by1, bx1, by2, bx2, bcl, bsc

    init = (scores0, zero_lane, zero_lane, zero_lane, zero_lane, zero_lane,
            zero_lane)
    _, by1, bx1, by2, bx2, bcl, bsc = jax.lax.fori_loop(
        0, _MAX_INST, body, init)
    valid = bsc > 0.0
    vf = valid.astype(jnp.float32)
    out_ref[...] = jnp.concatenate(
        [by1 * vf, bx1 * vf, by2 * vf, bx2 * vf, bcl * vf,
         jnp.where(valid, bsc, 0.0), zero_lane, zero_lane], axis=0)


def kernel(rois, probs, deltas, window):
    n, ncls = probs.shape
    grid = n // _TILE
    sc_b = -(-n // 512) * 512
    nbanks = -(-4 * ncls // _LANES)
    s2, c2, l2, windows = pl.pallas_call(
        _phase_a_kernel,
        grid=(grid,),
        in_specs=[
            pl.BlockSpec((_TILE, ncls), lambda i: (i, 0)),
            pl.BlockSpec((_TILE, nbanks * _LANES), lambda i: (i, 0)),
        ],
        out_specs=[
            pl.BlockSpec((_TILE, 1), lambda i: (i, 0)),
            pl.BlockSpec((_TILE, 1), lambda i: (i, 0)),
            pl.BlockSpec((_TILE, 1), lambda i: (i, 0)),
            pl.BlockSpec((_TILE, _LANES), lambda i: (i, 0)),
        ],
        out_shape=[
            jax.ShapeDtypeStruct((n, 1), jnp.float32),
            jax.ShapeDtypeStruct((n, 1), jnp.float32),
            jax.ShapeDtypeStruct((n, 1), jnp.int32),
            jax.ShapeDtypeStruct((sc_b, _LANES), jnp.float32),
        ],
    )(probs, deltas.reshape(n, 4 * ncls))

    lpad = jnp.pad(l2[:, 0], (0, sc_b - n))
    d4pl = _sc_extract(windows, lpad)

    n_pad = -(-n // _LANES) * _LANES
    rows = n_pad // _LANES
    pad = n_pad - n

    def plane(x):
        return jnp.pad(x[:, 0], (0, pad)).reshape(rows, _LANES)

    roisp = jnp.pad(rois, ((0, pad), (0, 0))).T.reshape(4, rows, _LANES)
    dselp = d4pl[:, :n_pad].reshape(4, rows, _LANES)

    det = pl.pallas_call(
        functools.partial(_phase_b_kernel, n),
        in_specs=[
            pl.BlockSpec(memory_space=pltpu.VMEM),
            pl.BlockSpec(memory_space=pltpu.VMEM),
            pl.BlockSpec(memory_space=pltpu.VMEM),
            pl.BlockSpec(memory_space=pltpu.VMEM),
            pl.BlockSpec(memory_space=pltpu.SMEM),
        ],
        out_shape=jax.ShapeDtypeStruct((8, _LANES), jnp.float32),
        scratch_shapes=[pltpu.VMEM((rows, _LANES), jnp.float32)] * 4,
    )(roisp, dselp, plane(c2), plane(s2), window)

    boxes = det[0:4, :_MAX_INST].T
    cls = det[4:5, :_MAX_INST].T
    sc = det[5:6, :_MAX_INST].T
    return jnp.concatenate([boxes, cls, sc], axis=1)
